# baseline (speedup 1.0000x reference)
import functools

import jax
import jax.numpy as jnp
from jax import lax
from jax.experimental import pallas as pl
from jax.experimental.pallas import tpu as pltpu
from jax.experimental.pallas import tpu_sc as plsc

GXR = 16
NB = 2


@functools.lru_cache(maxsize=None)
def _make_gather(batch, nf, V, D):
    B = batch * nf
    info = plsc.get_sparse_core_info()
    NC, NS = info.num_cores, info.num_subcores
    NW = NC * NS
    assert batch % (NW * GXR * NB) == 0
    r_per_w = batch // NW
    b_per_w = r_per_w * nf
    ng = r_per_w // GXR
    GCH = GXR * nf

    mesh = plsc.VectorSubcoreMesh(core_axis_name="c", subcore_axis_name="s")

    @functools.partial(
        pl.kernel,
        mesh=mesh,
        out_type=jax.ShapeDtypeStruct((batch, nf, D), jnp.float32),
        scratch_types=[
            pltpu.VMEM((b_per_w,), jnp.int32),
            pltpu.VMEM((NB, GCH, D), jnp.float32),
            pltpu.SemaphoreType.DMA,
            pltpu.SemaphoreType.DMA,
            pltpu.SemaphoreType.DMA,
        ],
        compiler_params=pltpu.CompilerParams(use_tc_tiling_on_sc=False),
    )
    def gather_kernel(idx_hbm, table_hbm, out_hbm, idx_v, rows, sem_g,
                      sem_o0, sem_o1):
        sem_o = [sem_o0, sem_o1]
        wid = lax.axis_index("s") * NC + lax.axis_index("c")
        rbase = wid * r_per_w
        ibase = wid * b_per_w
        pltpu.sync_copy(idx_hbm.at[pl.ds(ibase, b_per_w)], idx_v)

        def body(go, carry):
            for slot in range(NB):
                g = go * NB + slot

                @pl.when(go >= 1)
                def _():
                    pltpu.make_async_copy(
                        table_hbm.at[pl.ds(0, GCH)], rows.at[slot],
                        sem_o[slot],
                    ).wait()

                pltpu.async_copy(
                    table_hbm.at[idx_v.at[pl.ds(g * GCH, GCH)]],
                    rows.at[slot],
                    sem_g,
                ).wait()
                for j in range(GXR):
                    pltpu.async_copy(
                        rows.at[slot].at[pl.ds(j * nf, nf)],
                        out_hbm.at[rbase + g * GXR + j],
                        sem_o[slot],
                    )
            return carry

        lax.fori_loop(0, ng // NB, body, 0)

        for slot in range(NB):
            pltpu.make_async_copy(
                table_hbm.at[pl.ds(0, GCH)], rows.at[slot], sem_o[slot]
            ).wait()

    return gather_kernel


def kernel(x, weight):
    batch, nf = x.shape
    V, D = weight.shape
    idx = x.reshape(batch * nf)
    return _make_gather(batch, nf, V, D)(idx, weight)
